# baseline (speedup 1.0000x reference)
import functools
import math

import jax
import jax.numpy as jnp
from jax.experimental import pallas as pl
from jax.experimental.pallas import tpu as pltpu

NUM_CLASSES = 20
FIRST_N = 100
NUM_THRESH = 5
FEAT_DIM = 1024
NMS_FC = 128
POS_EMB = 64
ATT_FC = 16
GROUPS = 16
QK_DIM = 1024
N_PROP = 1000
DH = QK_DIM // GROUPS


def _pre_body(roi_ref, cls_ref, bbox_ref, prop_ref, wroi_ref, broi_ref,
              wrank_ref, brank_ref,
              roiemb_ref, rankfeat_ref, scores_ref, pbox_ref):
    roiemb_ref[...] = (
        jnp.dot(roi_ref[...], wroi_ref[...], preferred_element_type=jnp.float32)
        + broi_ref[...]
    )
    half = FEAT_DIM // 2
    fr = jax.lax.broadcasted_iota(jnp.int32, (FIRST_N, half), 1).astype(jnp.float32)
    dim_mat = jnp.exp(fr * ((2.0 / FEAT_DIM) * math.log(1000.0)))
    rank = jax.lax.broadcasted_iota(jnp.int32, (FIRST_N, half), 0).astype(jnp.float32)
    pos = rank / dim_mat
    rank_emb = jnp.concatenate([jnp.sin(pos), jnp.cos(pos)], axis=1)
    rankfeat_ref[...] = (
        jnp.dot(rank_emb, wrank_ref[...], preferred_element_type=jnp.float32)
        + brank_ref[...]
    )
    cls = cls_ref[...]
    m = jnp.max(cls, axis=-1, keepdims=True)
    e = jnp.exp(cls - m)
    s = jnp.sum(e, axis=-1, keepdims=True)
    scores_ref[...] = (e / s)[:, :NUM_CLASSES]
    p = prop_ref[...]
    x1 = p[:, 0:1] * 600.0
    y1 = p[:, 1:2] * 600.0
    x2 = x1 + p[:, 2:3] * 200.0 + 1.0
    y2 = y1 + p[:, 3:4] * 200.0 + 1.0
    w = x2 - x1
    h = y2 - y1
    cx = x1 + 0.5 * w
    cy = y1 + 0.5 * h
    b = bbox_ref[...]
    dx = b[:, 4:5] / 10.0
    dy = b[:, 5:6] / 10.0
    lim = math.log(1000.0 / 16.0)
    dw = jnp.minimum(b[:, 6:7] / 5.0, lim)
    dh = jnp.minimum(b[:, 7:8] / 5.0, lim)
    pcx = dx * w + cx
    pcy = dy * h + cy
    pw = jnp.exp(dw) * w
    ph = jnp.exp(dh) * h
    px1 = pcx - 0.5 * pw
    py1 = pcy - 0.5 * ph
    px2 = pcx + 0.5 * pw
    py2 = pcy + 0.5 * ph
    ws = jnp.maximum(px2 - px1, 1e-3)
    hs = jnp.maximum(py2 - py1, 1e-3)
    pbox_ref[...] = jnp.concatenate(
        [0.5 * (px1 + px2), 0.5 * (py1 + py2), jnp.log(ws), jnp.log(hs)], axis=1)


def _preprocess(roi_feat, cls_score, bbox_pred, proposal_boxes, W_roi, b_roi,
                W_rank, b_rank):
    return pl.pallas_call(
        _pre_body,
        out_shape=(
            jax.ShapeDtypeStruct((N_PROP, NMS_FC), jnp.float32),
            jax.ShapeDtypeStruct((FIRST_N, NMS_FC), jnp.float32),
            jax.ShapeDtypeStruct((N_PROP, NUM_CLASSES), jnp.float32),
            jax.ShapeDtypeStruct((N_PROP, 4), jnp.float32),
        ),
    )(roi_feat, cls_score, bbox_pred, proposal_boxes, W_roi,
      b_roi.reshape(1, NMS_FC), W_rank, b_rank.reshape(1, NMS_FC))


LOGEPS = math.log(1e-3)


def _freqs():
    return [100.0 * math.exp(-(f / 8.0) * math.log(1000.0)) for f in range(8)]


def _att_body(sroi_ref, rankf_ref, cstat_ref, rstat_ref, sscore_ref,
              wq_ref, bq_ref, wk_ref, bk_ref, wl_ref, bl_ref,
              wp_ref, mallw_ref, mallh_ref,
              wlg_ref, blg_ref, bp_ref, clampc_ref, out_ref):
    F = FIRST_N
    cf = _freqs()
    nms_emb = sroi_ref[0] + rankf_ref[...]
    q = jnp.dot(nms_emb, wq_ref[...], preferred_element_type=jnp.float32) + bq_ref[...]
    k = jnp.dot(nms_emb, wk_ref[...], preferred_element_type=jnp.float32) + bk_ref[...]

    cs = cstat_ref[0]
    rs = rstat_ref[0]
    cx_c, cy_c = cs[:, 0:1], cs[:, 1:2]
    lw_c, lh_c = cs[:, 2:3], cs[:, 3:4]
    cx_r = rs[0:1, :F]
    cy_r = rs[1:2, :F]
    lw_r = rs[2:3, :F]
    lh_r = rs[3:4, :F]

    def sep(u_c, u_r, mall_ref):
        args_a = jnp.concatenate([u_c * c for c in cf], axis=1)
        a_mat = jnp.concatenate([jnp.sin(args_a), jnp.cos(args_a)], axis=1)
        args_b = jnp.concatenate([u_r * c for c in cf], axis=0)
        bt = jnp.concatenate([jnp.sin(args_b), jnp.cos(args_b)], axis=0)
        p_all = jnp.dot(a_mat, mall_ref[...], preferred_element_type=jnp.float32)
        mask = (u_c - u_r) < LOGEPS
        return p_all, bt, mask

    pw_all, btw, mask_w = sep(lw_c, lw_r, mallw_ref)
    ph_all, bth, mask_h = sep(lh_c, lh_r, mallh_ref)

    def prep_xy(c_c, c_r, b_c, pbase):
        a = jnp.log(jnp.abs(c_c - c_r))
        mask = a < (LOGEPS + b_c)
        smats, cmats, coefa, coefb = [], [], [], []
        for f in range(8):
            arg = a * cf[f]
            smats.append(jnp.sin(arg))
            cmats.append(jnp.cos(arg))
            ws = wp_ref[pbase + f:pbase + f + 1, :]
            wc = wp_ref[pbase + 8 + f:pbase + 9 + f, :]
            cb = jnp.cos(b_c * cf[f])
            sb = jnp.sin(b_c * cf[f])
            coefa.append(cb * ws + sb * wc)
            coefb.append(cb * wc - sb * ws)
        return smats, cmats, coefa, coefb, mask

    sx, cxm, ax, bx, mask_x = prep_xy(cx_c, cx_r, lw_c, 0)
    sy, cym, ay, by, mask_y = prep_xy(cy_c, cy_r, lh_c, 16)

    acc_out = jnp.zeros((F, NMS_FC), jnp.float32)
    inv_sqrt = 1.0 / math.sqrt(float(DH))
    for h in range(GROUPS):
        dxp = jnp.zeros((F, F), jnp.float32)
        dyp = jnp.zeros((F, F), jnp.float32)
        for f in range(8):
            dxp = dxp + sx[f] * ax[f][:, h:h + 1] + cxm[f] * bx[f][:, h:h + 1]
            dyp = dyp + sy[f] * ay[f][:, h:h + 1] + cym[f] * by[f][:, h:h + 1]
        dxp = jnp.where(mask_x, clampc_ref[0, h], dxp)
        dyp = jnp.where(mask_y, clampc_ref[1, h], dyp)
        dwp = jnp.dot(pw_all[:, h * 16:(h + 1) * 16], btw,
                      preferred_element_type=jnp.float32)
        dwp = jnp.where(mask_w, clampc_ref[2, h], dwp)
        dhp = jnp.dot(ph_all[:, h * 16:(h + 1) * 16], bth,
                      preferred_element_type=jnp.float32)
        dhp = jnp.where(mask_h, clampc_ref[3, h], dhp)
        aff_w = jnp.maximum(dxp + dyp + dwp + dhp + bp_ref[0, h], 0.0)
        q_h = q[:, h * DH:(h + 1) * DH]
        k_h = k[:, h * DH:(h + 1) * DH]
        aff = jax.lax.dot_general(
            q_h, k_h, (((1,), (1,)), ((), ())),
            preferred_element_type=jnp.float32) * inv_sqrt
        weighted = jnp.log(jnp.maximum(aff_w, 1e-6)) + aff
        m = jnp.max(weighted, axis=-1, keepdims=True)
        e = jnp.exp(weighted - m)
        att = e / jnp.sum(e, axis=-1, keepdims=True)
        o_h = jnp.dot(att, nms_emb, preferred_element_type=jnp.float32)
        acc_out = acc_out + jnp.dot(o_h, wl_ref[h * NMS_FC:(h + 1) * NMS_FC, :],
                                    preferred_element_type=jnp.float32)

    all_feat = jnp.maximum(nms_emb + acc_out + bl_ref[...], 0.0)
    logit = jnp.dot(all_feat, wlg_ref[...], preferred_element_type=jnp.float32) + blg_ref[...]
    cond = 1.0 / (1.0 + jnp.exp(-logit))
    out_ref[0] = sscore_ref[0] * cond


def _attention(sroi, rank_feat, cstat, rstat, sscore, Wq, bq, Wk, bk, Wl, bl,
               Wp, mall_w, mall_h, W_logit, b_logit, bp, clampc):
    C = NUM_CLASSES
    grid = (C,)

    def cblk(shape):
        return pl.BlockSpec((1,) + shape, lambda c: (c, 0, 0))

    def whole(shape):
        nd = len(shape)
        return pl.BlockSpec(shape, lambda c: (0,) * nd)

    return pl.pallas_call(
        _att_body,
        grid=grid,
        in_specs=[
            cblk((FIRST_N, NMS_FC)),
            whole((FIRST_N, NMS_FC)),
            cblk((FIRST_N, 4)),
            cblk((8, 128)),
            cblk((FIRST_N, 1)),
            whole((NMS_FC, QK_DIM)),
            whole((1, QK_DIM)),
            whole((NMS_FC, QK_DIM)),
            whole((1, QK_DIM)),
            whole((GROUPS * NMS_FC, NMS_FC)),
            whole((1, NMS_FC)),
            whole((POS_EMB, ATT_FC)),
            whole((16, 256)),
            whole((16, 256)),
            whole((NMS_FC, NUM_THRESH)),
            whole((1, NUM_THRESH)),
            pl.BlockSpec(memory_space=pltpu.SMEM),
            pl.BlockSpec(memory_space=pltpu.SMEM),
        ],
        out_specs=cblk((FIRST_N, NUM_THRESH)),
        out_shape=jax.ShapeDtypeStruct((C, FIRST_N, NUM_THRESH), jnp.float32),
    )(sroi, rank_feat, cstat, rstat, sscore, Wq, bq.reshape(1, QK_DIM),
      Wk, bk.reshape(1, QK_DIM), Wl, bl.reshape(1, NMS_FC), Wp,
      mall_w, mall_h, W_logit, b_logit.reshape(1, NUM_THRESH), bp, clampc)


def _weight_transforms(Wp, bp):
    def mall(pbase):
        ws = Wp[pbase:pbase + 8, :]
        wc = Wp[pbase + 8:pbase + 16, :]
        blocks = []
        for h in range(GROUPS):
            m11 = jnp.diag(wc[:, h])
            m12 = jnp.diag(ws[:, h])
            top = jnp.concatenate([m11, m12], axis=1)
            bot = jnp.concatenate([-m12, m11], axis=1)
            blocks.append(jnp.concatenate([top, bot], axis=0))
        return jnp.concatenate(blocks, axis=1)

    cf = jnp.asarray(_freqs())
    args = LOGEPS * cf
    s, c = jnp.sin(args), jnp.cos(args)
    clampc = []
    for p in range(4):
        ws = Wp[p * 16:p * 16 + 8, :]
        wc = Wp[p * 16 + 8:p * 16 + 16, :]
        clampc.append(s @ ws + c @ wc)
    clampc = jnp.stack(clampc, axis=0)
    return mall(32), mall(48), clampc, bp.reshape(1, GROUPS)


def kernel(roi_feat, cls_score, bbox_pred, proposal_boxes, W_roi, b_roi,
           W_rank, b_rank, W_logit, b_logit, Wp, bp, Wq, bq, Wk, bk, Wl, bl,
           num_boxes):
    roi_emb, rank_feat, scores, pstats = _preprocess(
        roi_feat, cls_score, bbox_pred, proposal_boxes, W_roi, b_roi,
        W_rank, b_rank)

    vals, idx = jax.lax.top_k(scores.T, FIRST_N)
    sroi = roi_emb[idx]
    cstat = pstats[idx]
    rstat = jnp.zeros((NUM_CLASSES, 8, 128), jnp.float32)
    rstat = rstat.at[:, :4, :FIRST_N].set(jnp.transpose(cstat, (0, 2, 1)))
    sscore = vals[..., None]

    mall_w, mall_h, clampc, bp_s = _weight_transforms(Wp, bp)
    out = _attention(sroi, rank_feat, cstat, rstat, sscore, Wq, bq, Wk, bk,
                     Wl, bl, Wp, mall_w, mall_h, W_logit, b_logit, bp_s,
                     clampc)
    return jnp.transpose(out, (1, 0, 2))[None]


# speedup vs baseline: 1.6065x; 1.6065x over previous
import functools
import math

import jax
import jax.numpy as jnp
from jax.experimental import pallas as pl
from jax.experimental.pallas import tpu as pltpu

NUM_CLASSES = 20
FIRST_N = 100
NUM_THRESH = 5
FEAT_DIM = 1024
NMS_FC = 128
POS_EMB = 64
ATT_FC = 16
GROUPS = 16
QK_DIM = 1024
N_PROP = 1000
DH = QK_DIM // GROUPS


def _pre_body(roi_ref, cls_ref, bbox_ref, prop_ref, wroi_ref, broi_ref,
              wrank_ref, brank_ref,
              roiemb_ref, rankfeat_ref, scores_ref, pbox_ref):
    roiemb_ref[...] = (
        jnp.dot(roi_ref[...], wroi_ref[...], preferred_element_type=jnp.float32)
        + broi_ref[...]
    )
    half = FEAT_DIM // 2
    fr = jax.lax.broadcasted_iota(jnp.int32, (FIRST_N, half), 1).astype(jnp.float32)
    dim_mat = jnp.exp(fr * ((2.0 / FEAT_DIM) * math.log(1000.0)))
    rank = jax.lax.broadcasted_iota(jnp.int32, (FIRST_N, half), 0).astype(jnp.float32)
    pos = rank / dim_mat
    rank_emb = jnp.concatenate([jnp.sin(pos), jnp.cos(pos)], axis=1)
    rankfeat_ref[...] = (
        jnp.dot(rank_emb, wrank_ref[...], preferred_element_type=jnp.float32)
        + brank_ref[...]
    )
    cls = cls_ref[...]
    m = jnp.max(cls, axis=-1, keepdims=True)
    e = jnp.exp(cls - m)
    s = jnp.sum(e, axis=-1, keepdims=True)
    scores_ref[...] = (e / s)[:, :NUM_CLASSES]
    p = prop_ref[...]
    x1 = p[:, 0:1] * 600.0
    y1 = p[:, 1:2] * 600.0
    x2 = x1 + p[:, 2:3] * 200.0 + 1.0
    y2 = y1 + p[:, 3:4] * 200.0 + 1.0
    w = x2 - x1
    h = y2 - y1
    cx = x1 + 0.5 * w
    cy = y1 + 0.5 * h
    b = bbox_ref[...]
    dx = b[:, 4:5] / 10.0
    dy = b[:, 5:6] / 10.0
    lim = math.log(1000.0 / 16.0)
    dw = jnp.minimum(b[:, 6:7] / 5.0, lim)
    dh = jnp.minimum(b[:, 7:8] / 5.0, lim)
    pcx = dx * w + cx
    pcy = dy * h + cy
    pw = jnp.exp(dw) * w
    ph = jnp.exp(dh) * h
    px1 = pcx - 0.5 * pw
    py1 = pcy - 0.5 * ph
    px2 = pcx + 0.5 * pw
    py2 = pcy + 0.5 * ph
    ws = jnp.maximum(px2 - px1, 1e-3)
    hs = jnp.maximum(py2 - py1, 1e-3)
    pbox_ref[...] = jnp.concatenate(
        [0.5 * (px1 + px2), 0.5 * (py1 + py2), jnp.log(ws), jnp.log(hs)], axis=1)


def _preprocess(roi_feat, cls_score, bbox_pred, proposal_boxes, W_roi, b_roi,
                W_rank, b_rank):
    return pl.pallas_call(
        _pre_body,
        out_shape=(
            jax.ShapeDtypeStruct((N_PROP, NMS_FC), jnp.float32),
            jax.ShapeDtypeStruct((FIRST_N, NMS_FC), jnp.float32),
            jax.ShapeDtypeStruct((N_PROP, NUM_CLASSES), jnp.float32),
            jax.ShapeDtypeStruct((N_PROP, 4), jnp.float32),
        ),
    )(roi_feat, cls_score, bbox_pred, proposal_boxes, W_roi,
      b_roi.reshape(1, NMS_FC), W_rank, b_rank.reshape(1, NMS_FC))


LOGEPS = math.log(1e-3)


def _freqs():
    return [100.0 * math.exp(-(f / 8.0) * math.log(1000.0)) for f in range(8)]


def _att_body(sroi_ref, rankf_ref, cstat_ref, rstat_ref, sscore_ref,
              wq_ref, bq_ref, wk_ref, bk_ref, wl_ref, bl_ref,
              wp_ref, mallw_ref, mallh_ref,
              wlg_ref, blg_ref, bp_ref, clampc_ref, out_ref):
    F = FIRST_N
    cf = _freqs()
    nms_emb = sroi_ref[0] + rankf_ref[...]
    q = jnp.dot(nms_emb, wq_ref[...], preferred_element_type=jnp.float32) + bq_ref[...]
    k = jnp.dot(nms_emb, wk_ref[...], preferred_element_type=jnp.float32) + bk_ref[...]

    cs = cstat_ref[0]
    rs = rstat_ref[0]
    cx_c, cy_c = cs[:, 0:1], cs[:, 1:2]
    lw_c, lh_c = cs[:, 2:3], cs[:, 3:4]
    cx_r = rs[0:1, :F]
    cy_r = rs[1:2, :F]
    lw_r = rs[2:3, :F]
    lh_r = rs[3:4, :F]


    def sep(u_c, u_r, mall_ref):
        args_a = jnp.concatenate([u_c * c for c in cf], axis=1)
        a_mat = jnp.concatenate([jnp.sin(args_a), jnp.cos(args_a)], axis=1)
        args_b = jnp.concatenate([u_r * c for c in cf], axis=0)
        bt = jnp.concatenate([jnp.sin(args_b), jnp.cos(args_b)], axis=0)
        p_all = jnp.dot(a_mat, mall_ref[...], preferred_element_type=jnp.float32)
        mask = (u_r - u_c) < LOGEPS
        return p_all, bt, mask

    pw_all, btw, mask_w = sep(lw_c, lw_r, mallw_ref)
    ph_all, bth, mask_h = sep(lh_c, lh_r, mallh_ref)

    def prep_xy(c_c, c_r, b_r, pbase):
        a = jnp.log(jnp.abs(c_c - c_r))
        mask = a < (LOGEPS + b_r)
        smats, cmats, coefa, coefb = [], [], [], []
        for f in range(8):
            arg = a * cf[f]
            smats.append(jnp.sin(arg))
            cmats.append(jnp.cos(arg))
            cb = jnp.cos(b_r * cf[f])
            sb = jnp.sin(b_r * cf[f])
            coefa.append(cb)
            coefb.append(sb)
        return smats, cmats, coefa, coefb, mask

    sx, cxm, cbx, sbx, mask_x = prep_xy(cx_c, cx_r, lw_r, 0)
    sy, cym, cby, sby, mask_y = prep_xy(cy_c, cy_r, lh_r, 16)

    acc_out = jnp.zeros((F, NMS_FC), jnp.float32)
    inv_sqrt = 1.0 / math.sqrt(float(DH))
    for h in range(GROUPS):
        dxp = jnp.zeros((F, F), jnp.float32)
        dyp = jnp.zeros((F, F), jnp.float32)
        for f in range(8):
            wsx = wp_ref[f, h]
            wcx = wp_ref[8 + f, h]
            rax = cbx[f] * wsx + sbx[f] * wcx
            rbx = cbx[f] * wcx - sbx[f] * wsx
            dxp = dxp + sx[f] * rax + cxm[f] * rbx
            wsy = wp_ref[16 + f, h]
            wcy = wp_ref[24 + f, h]
            ray = cby[f] * wsy + sby[f] * wcy
            rby = cby[f] * wcy - sby[f] * wsy
            dyp = dyp + sy[f] * ray + cym[f] * rby
        dxp = jnp.where(mask_x, clampc_ref[0, h], dxp)
        dyp = jnp.where(mask_y, clampc_ref[1, h], dyp)
        dwp = jnp.dot(pw_all[:, h * 16:(h + 1) * 16], btw,
                      preferred_element_type=jnp.float32)
        dwp = jnp.where(mask_w, clampc_ref[2, h], dwp)
        dhp = jnp.dot(ph_all[:, h * 16:(h + 1) * 16], bth,
                      preferred_element_type=jnp.float32)
        dhp = jnp.where(mask_h, clampc_ref[3, h], dhp)
        aff_w = jnp.maximum(dxp + dyp + dwp + dhp + bp_ref[0, h], 0.0)
        q_h = q[:, h * DH:(h + 1) * DH]
        k_h = k[:, h * DH:(h + 1) * DH]
        aff_t = jax.lax.dot_general(
            k_h, q_h, (((1,), (1,)), ((), ())),
            preferred_element_type=jnp.float32) * inv_sqrt
        weighted = jnp.log(jnp.maximum(aff_w, 1e-6)) + aff_t
        m = jnp.max(weighted, axis=0, keepdims=True)
        e = jnp.exp(weighted - m)
        att_t = e / jnp.sum(e, axis=0, keepdims=True)
        o_h = jax.lax.dot_general(
            att_t, nms_emb, (((0,), (0,)), ((), ())),
            preferred_element_type=jnp.float32)
        acc_out = acc_out + jnp.dot(o_h, wl_ref[h * NMS_FC:(h + 1) * NMS_FC, :],
                                    preferred_element_type=jnp.float32)

    all_feat = jnp.maximum(nms_emb + acc_out + bl_ref[...], 0.0)
    logit = jnp.dot(all_feat, wlg_ref[...], preferred_element_type=jnp.float32) + blg_ref[...]
    cond = 1.0 / (1.0 + jnp.exp(-logit))
    out_ref[0] = sscore_ref[0] * cond


def _attention(sroi, rank_feat, cstat, rstat, sscore, Wq, bq, Wk, bk, Wl, bl,
               Wp, mall_w, mall_h, W_logit, b_logit, bp, clampc):
    C = NUM_CLASSES
    grid = (C,)

    def cblk(shape):
        return pl.BlockSpec((1,) + shape, lambda c: (c, 0, 0))

    def whole(shape):
        nd = len(shape)
        return pl.BlockSpec(shape, lambda c: (0,) * nd)

    return pl.pallas_call(
        _att_body,
        grid=grid,
        in_specs=[
            cblk((FIRST_N, NMS_FC)),
            whole((FIRST_N, NMS_FC)),
            cblk((FIRST_N, 4)),
            cblk((8, 128)),
            cblk((FIRST_N, 1)),
            whole((NMS_FC, QK_DIM)),
            whole((1, QK_DIM)),
            whole((NMS_FC, QK_DIM)),
            whole((1, QK_DIM)),
            whole((GROUPS * NMS_FC, NMS_FC)),
            whole((1, NMS_FC)),
            pl.BlockSpec(memory_space=pltpu.SMEM),
            whole((16, 256)),
            whole((16, 256)),
            whole((NMS_FC, NUM_THRESH)),
            whole((1, NUM_THRESH)),
            pl.BlockSpec(memory_space=pltpu.SMEM),
            pl.BlockSpec(memory_space=pltpu.SMEM),
        ],
        out_specs=cblk((FIRST_N, NUM_THRESH)),
        out_shape=jax.ShapeDtypeStruct((C, FIRST_N, NUM_THRESH), jnp.float32),
    )(sroi, rank_feat, cstat, rstat, sscore, Wq, bq.reshape(1, QK_DIM),
      Wk, bk.reshape(1, QK_DIM), Wl, bl.reshape(1, NMS_FC), Wp,
      mall_w, mall_h, W_logit, b_logit.reshape(1, NUM_THRESH), bp, clampc)


def _weight_transforms(Wp, bp):
    def mall(pbase):
        ws = Wp[pbase:pbase + 8, :]
        wc = Wp[pbase + 8:pbase + 16, :]
        blocks = []
        for h in range(GROUPS):
            m11 = jnp.diag(wc[:, h])
            m12 = jnp.diag(ws[:, h])
            top = jnp.concatenate([m11, -m12], axis=1)
            bot = jnp.concatenate([m12, m11], axis=1)
            blocks.append(jnp.concatenate([top, bot], axis=0))
        return jnp.concatenate(blocks, axis=1)

    cf = jnp.asarray(_freqs())
    args = LOGEPS * cf
    s, c = jnp.sin(args), jnp.cos(args)
    clampc = []
    for p in range(4):
        ws = Wp[p * 16:p * 16 + 8, :]
        wc = Wp[p * 16 + 8:p * 16 + 16, :]
        clampc.append(s @ ws + c @ wc)
    clampc = jnp.stack(clampc, axis=0)
    return mall(32), mall(48), clampc, bp.reshape(1, GROUPS)


def kernel(roi_feat, cls_score, bbox_pred, proposal_boxes, W_roi, b_roi,
           W_rank, b_rank, W_logit, b_logit, Wp, bp, Wq, bq, Wk, bk, Wl, bl,
           num_boxes):
    roi_emb, rank_feat, scores, pstats = _preprocess(
        roi_feat, cls_score, bbox_pred, proposal_boxes, W_roi, b_roi,
        W_rank, b_rank)

    vals, idx = jax.lax.top_k(scores.T, FIRST_N)
    sroi = roi_emb[idx]
    cstat = pstats[idx]
    rstat = jnp.zeros((NUM_CLASSES, 8, 128), jnp.float32)
    rstat = rstat.at[:, :4, :FIRST_N].set(jnp.transpose(cstat, (0, 2, 1)))
    sscore = vals[..., None]

    mall_w, mall_h, clampc, bp_s = _weight_transforms(Wp, bp)
    out = _attention(sroi, rank_feat, cstat, rstat, sscore, Wq, bq, Wk, bk,
                     Wl, bl, Wp, mall_w, mall_h, W_logit, b_logit, bp_s,
                     clampc)
    return jnp.transpose(out, (1, 0, 2))[None]


# speedup vs baseline: 1.9735x; 1.2284x over previous
import functools
import math

import jax
import jax.numpy as jnp
from jax.experimental import pallas as pl
from jax.experimental.pallas import tpu as pltpu

NUM_CLASSES = 20
FIRST_N = 100
NUM_THRESH = 5
FEAT_DIM = 1024
NMS_FC = 128
POS_EMB = 64
ATT_FC = 16
GROUPS = 16
QK_DIM = 1024
N_PROP = 1000
DH = QK_DIM // GROUPS


def _pre_body(roi_ref, cls_ref, bbox_ref, prop_ref, wroi_ref, broi_ref,
              wrank_ref, brank_ref,
              roiemb_ref, rankfeat_ref, scores_ref, pbox_ref):
    roiemb_ref[...] = (
        jnp.dot(roi_ref[...], wroi_ref[...], preferred_element_type=jnp.float32)
        + broi_ref[...]
    )
    half = FEAT_DIM // 2
    fr = jax.lax.broadcasted_iota(jnp.int32, (FIRST_N, half), 1).astype(jnp.float32)
    dim_mat = jnp.exp(fr * ((2.0 / FEAT_DIM) * math.log(1000.0)))
    rank = jax.lax.broadcasted_iota(jnp.int32, (FIRST_N, half), 0).astype(jnp.float32)
    pos = rank / dim_mat
    rank_emb = jnp.concatenate([jnp.sin(pos), jnp.cos(pos)], axis=1)
    rankfeat_ref[...] = (
        jnp.dot(rank_emb, wrank_ref[...], preferred_element_type=jnp.float32)
        + brank_ref[...]
    )
    cls = cls_ref[...]
    m = jnp.max(cls, axis=-1, keepdims=True)
    e = jnp.exp(cls - m)
    s = jnp.sum(e, axis=-1, keepdims=True)
    scores_ref[...] = (e / s)[:, :NUM_CLASSES]
    p = prop_ref[...]
    x1 = p[:, 0:1] * 600.0
    y1 = p[:, 1:2] * 600.0
    x2 = x1 + p[:, 2:3] * 200.0 + 1.0
    y2 = y1 + p[:, 3:4] * 200.0 + 1.0
    w = x2 - x1
    h = y2 - y1
    cx = x1 + 0.5 * w
    cy = y1 + 0.5 * h
    b = bbox_ref[...]
    dx = b[:, 4:5] / 10.0
    dy = b[:, 5:6] / 10.0
    lim = math.log(1000.0 / 16.0)
    dw = jnp.minimum(b[:, 6:7] / 5.0, lim)
    dh = jnp.minimum(b[:, 7:8] / 5.0, lim)
    pcx = dx * w + cx
    pcy = dy * h + cy
    pw = jnp.exp(dw) * w
    ph = jnp.exp(dh) * h
    px1 = pcx - 0.5 * pw
    py1 = pcy - 0.5 * ph
    px2 = pcx + 0.5 * pw
    py2 = pcy + 0.5 * ph
    ws = jnp.maximum(px2 - px1, 1e-3)
    hs = jnp.maximum(py2 - py1, 1e-3)
    pbox_ref[...] = jnp.concatenate(
        [0.5 * (px1 + px2), 0.5 * (py1 + py2), jnp.log(ws), jnp.log(hs)], axis=1)


def _preprocess(roi_feat, cls_score, bbox_pred, proposal_boxes, W_roi, b_roi,
                W_rank, b_rank):
    return pl.pallas_call(
        _pre_body,
        out_shape=(
            jax.ShapeDtypeStruct((N_PROP, NMS_FC), jnp.float32),
            jax.ShapeDtypeStruct((FIRST_N, NMS_FC), jnp.float32),
            jax.ShapeDtypeStruct((N_PROP, NUM_CLASSES), jnp.float32),
            jax.ShapeDtypeStruct((N_PROP, 4), jnp.float32),
        ),
    )(roi_feat, cls_score, bbox_pred, proposal_boxes, W_roi,
      b_roi.reshape(1, NMS_FC), W_rank, b_rank.reshape(1, NMS_FC))


LOGEPS = math.log(1e-3)


def _freqs():
    return [100.0 * math.exp(-(f / 8.0) * math.log(1000.0)) for f in range(8)]


_SIN_C = (6.2831834663762, -41.34148035624615, 81.59765787614153,
          -76.59492821657145, 41.26992956767051, -12.372494818441739)
_COS_C = (0.9999999922902976, -19.739205554044485, 64.93917223259542,
          -85.45116579292134, 60.176230338873005, -26.000527873768437,
          6.57561164274851)


def _fast_sincos(t):
    r = t - jnp.floor(t + 0.5)
    x2 = r * r
    s = _SIN_C[5]
    for c in _SIN_C[4::-1]:
        s = s * x2 + c
    s = s * r
    c_ = _COS_C[6]
    for c in _COS_C[5::-1]:
        c_ = c_ * x2 + c
    return s, c_


def _att_body(sroi_ref, rankf_ref, cstat_ref, rstat_ref, sscore_ref,
              wq_ref, bq_ref, wk_ref, bk_ref, wl_ref, bl_ref,
              wp_ref, mallw_ref, mallh_ref,
              wlg_ref, blg_ref, bp_ref, clampc_ref, out_ref):
    F = FIRST_N
    cf = _freqs()
    nms_emb = sroi_ref[0] + rankf_ref[...]
    q = jnp.dot(nms_emb, wq_ref[...], preferred_element_type=jnp.float32) + bq_ref[...]
    k = jnp.dot(nms_emb, wk_ref[...], preferred_element_type=jnp.float32) + bk_ref[...]

    cs = cstat_ref[0]
    rs = rstat_ref[0]
    cx_c, cy_c = cs[:, 0:1], cs[:, 1:2]
    lw_c, lh_c = cs[:, 2:3], cs[:, 3:4]
    cx_r = rs[0:1, :F]
    cy_r = rs[1:2, :F]
    lw_r = rs[2:3, :F]
    lh_r = rs[3:4, :F]


    def sep(u_c, u_r, mall_ref):
        inv2pi = 1.0 / (2.0 * math.pi)
        ta = jnp.concatenate([u_c * (c * inv2pi) for c in cf], axis=1)
        sa, ca = _fast_sincos(ta)
        a_mat = jnp.concatenate([sa, ca], axis=1)
        tb = jnp.concatenate([u_r * (c * inv2pi) for c in cf], axis=0)
        sb, cb = _fast_sincos(tb)
        bt = jnp.concatenate([sb, cb], axis=0)
        p_all = jnp.dot(a_mat, mall_ref[...], preferred_element_type=jnp.float32)
        mask = (u_r - u_c) < LOGEPS
        return p_all, bt, mask

    pw_all, btw, mask_w = sep(lw_c, lw_r, mallw_ref)
    ph_all, bth, mask_h = sep(lh_c, lh_r, mallh_ref)

    def prep_xy(c_c, c_r, b_r, pbase):
        a = jnp.log(jnp.abs(c_c - c_r))
        mask = a < (LOGEPS + b_r)
        smats, cmats, coefa, coefb = [], [], [], []
        inv2pi = 1.0 / (2.0 * math.pi)
        for f in range(8):
            s, c = _fast_sincos(a * (cf[f] * inv2pi))
            smats.append(s)
            cmats.append(c)
            sb, cb = _fast_sincos(b_r * (cf[f] * inv2pi))
            coefa.append(cb)
            coefb.append(sb)
        return smats, cmats, coefa, coefb, mask

    sx, cxm, cbx, sbx, mask_x = prep_xy(cx_c, cx_r, lw_r, 0)
    sy, cym, cby, sby, mask_y = prep_xy(cy_c, cy_r, lh_r, 16)

    acc_out = jnp.zeros((F, NMS_FC), jnp.float32)
    inv_sqrt = 1.0 / math.sqrt(float(DH))
    for h in range(GROUPS):
        dxp = jnp.zeros((F, F), jnp.float32)
        dyp = jnp.zeros((F, F), jnp.float32)
        for f in range(8):
            wsx = wp_ref[f, h]
            wcx = wp_ref[8 + f, h]
            rax = cbx[f] * wsx + sbx[f] * wcx
            rbx = cbx[f] * wcx - sbx[f] * wsx
            dxp = dxp + sx[f] * rax + cxm[f] * rbx
            wsy = wp_ref[16 + f, h]
            wcy = wp_ref[24 + f, h]
            ray = cby[f] * wsy + sby[f] * wcy
            rby = cby[f] * wcy - sby[f] * wsy
            dyp = dyp + sy[f] * ray + cym[f] * rby
        dxp = jnp.where(mask_x, clampc_ref[0, h], dxp)
        dyp = jnp.where(mask_y, clampc_ref[1, h], dyp)
        dwp = jnp.dot(pw_all[:, h * 16:(h + 1) * 16], btw,
                      preferred_element_type=jnp.float32)
        dwp = jnp.where(mask_w, clampc_ref[2, h], dwp)
        dhp = jnp.dot(ph_all[:, h * 16:(h + 1) * 16], bth,
                      preferred_element_type=jnp.float32)
        dhp = jnp.where(mask_h, clampc_ref[3, h], dhp)
        aff_w = jnp.maximum(dxp + dyp + dwp + dhp + bp_ref[0, h], 0.0)
        q_h = q[:, h * DH:(h + 1) * DH]
        k_h = k[:, h * DH:(h + 1) * DH]
        aff_t = jax.lax.dot_general(
            k_h, q_h, (((1,), (1,)), ((), ())),
            preferred_element_type=jnp.float32) * inv_sqrt
        weighted = jnp.log(jnp.maximum(aff_w, 1e-6)) + aff_t
        m = jnp.max(weighted, axis=0, keepdims=True)
        e = jnp.exp(weighted - m)
        att_t = e / jnp.sum(e, axis=0, keepdims=True)
        o_h = jax.lax.dot_general(
            att_t, nms_emb, (((0,), (0,)), ((), ())),
            preferred_element_type=jnp.float32)
        acc_out = acc_out + jnp.dot(o_h, wl_ref[h * NMS_FC:(h + 1) * NMS_FC, :],
                                    preferred_element_type=jnp.float32)

    all_feat = jnp.maximum(nms_emb + acc_out + bl_ref[...], 0.0)
    logit = jnp.dot(all_feat, wlg_ref[...], preferred_element_type=jnp.float32) + blg_ref[...]
    cond = 1.0 / (1.0 + jnp.exp(-logit))
    out_ref[0] = sscore_ref[0] * cond


def _attention(sroi, rank_feat, cstat, rstat, sscore, Wq, bq, Wk, bk, Wl, bl,
               Wp, mall_w, mall_h, W_logit, b_logit, bp, clampc):
    C = NUM_CLASSES
    grid = (C,)

    def cblk(shape):
        return pl.BlockSpec((1,) + shape, lambda c: (c, 0, 0))

    def whole(shape):
        nd = len(shape)
        return pl.BlockSpec(shape, lambda c: (0,) * nd)

    return pl.pallas_call(
        _att_body,
        grid=grid,
        in_specs=[
            cblk((FIRST_N, NMS_FC)),
            whole((FIRST_N, NMS_FC)),
            cblk((FIRST_N, 4)),
            cblk((8, 128)),
            cblk((FIRST_N, 1)),
            whole((NMS_FC, QK_DIM)),
            whole((1, QK_DIM)),
            whole((NMS_FC, QK_DIM)),
            whole((1, QK_DIM)),
            whole((GROUPS * NMS_FC, NMS_FC)),
            whole((1, NMS_FC)),
            pl.BlockSpec(memory_space=pltpu.SMEM),
            whole((16, 256)),
            whole((16, 256)),
            whole((NMS_FC, NUM_THRESH)),
            whole((1, NUM_THRESH)),
            pl.BlockSpec(memory_space=pltpu.SMEM),
            pl.BlockSpec(memory_space=pltpu.SMEM),
        ],
        out_specs=cblk((FIRST_N, NUM_THRESH)),
        out_shape=jax.ShapeDtypeStruct((C, FIRST_N, NUM_THRESH), jnp.float32),
    )(sroi, rank_feat, cstat, rstat, sscore, Wq, bq.reshape(1, QK_DIM),
      Wk, bk.reshape(1, QK_DIM), Wl, bl.reshape(1, NMS_FC), Wp,
      mall_w, mall_h, W_logit, b_logit.reshape(1, NUM_THRESH), bp, clampc)


def _weight_transforms(Wp, bp):
    def mall(pbase):
        ws = Wp[pbase:pbase + 8, :]
        wc = Wp[pbase + 8:pbase + 16, :]
        blocks = []
        for h in range(GROUPS):
            m11 = jnp.diag(wc[:, h])
            m12 = jnp.diag(ws[:, h])
            top = jnp.concatenate([m11, -m12], axis=1)
            bot = jnp.concatenate([m12, m11], axis=1)
            blocks.append(jnp.concatenate([top, bot], axis=0))
        return jnp.concatenate(blocks, axis=1)

    cf = jnp.asarray(_freqs())
    args = LOGEPS * cf
    s, c = jnp.sin(args), jnp.cos(args)
    clampc = []
    for p in range(4):
        ws = Wp[p * 16:p * 16 + 8, :]
        wc = Wp[p * 16 + 8:p * 16 + 16, :]
        clampc.append(s @ ws + c @ wc)
    clampc = jnp.stack(clampc, axis=0)
    return mall(32), mall(48), clampc, bp.reshape(1, GROUPS)


def kernel(roi_feat, cls_score, bbox_pred, proposal_boxes, W_roi, b_roi,
           W_rank, b_rank, W_logit, b_logit, Wp, bp, Wq, bq, Wk, bk, Wl, bl,
           num_boxes):
    roi_emb, rank_feat, scores, pstats = _preprocess(
        roi_feat, cls_score, bbox_pred, proposal_boxes, W_roi, b_roi,
        W_rank, b_rank)

    vals, idx = jax.lax.top_k(scores.T, FIRST_N)
    sroi = roi_emb[idx]
    cstat = pstats[idx]
    rstat = jnp.zeros((NUM_CLASSES, 8, 128), jnp.float32)
    rstat = rstat.at[:, :4, :FIRST_N].set(jnp.transpose(cstat, (0, 2, 1)))
    sscore = vals[..., None]

    mall_w, mall_h, clampc, bp_s = _weight_transforms(Wp, bp)
    out = _attention(sroi, rank_feat, cstat, rstat, sscore, Wq, bq, Wk, bk,
                     Wl, bl, Wp, mall_w, mall_h, W_logit, b_logit, bp_s,
                     clampc)
    return jnp.transpose(out, (1, 0, 2))[None]
